# baseline (speedup 1.0000x reference)
import functools

import jax
import jax.numpy as jnp
from jax import lax
from jax.experimental import pallas as pl
from jax.experimental.pallas import tpu as pltpu
from jax.experimental.pallas import tpu_sc as plsc

VOCAB_NO_POS = 999001
EMB = 64
SCALE = 8.0
LANES = 16
SEQ = 200
BATCH = 4096
C = 128

_info = plsc.get_sparse_core_info()
NC = _info.num_cores
NS = _info.num_subcores
NW = NC * NS

VOCAB_BLK = 2048


def _prep_body(t_ref, o_ref):
    f = lax.broadcasted_iota(jnp.int32, (EMB, EMB), 0)
    e = lax.broadcasted_iota(jnp.int32, (EMB, EMB), 1)
    eye8 = jnp.where(f == e, SCALE, 0.0).astype(jnp.float32)
    o_ref[:, 0:EMB] = lax.dot_general(
        t_ref[...], eye8, (((0,), (0,)), ((), ())),
        preferred_element_type=jnp.float32,
    )


@jax.jit
def _prep_table(table_t):
    n_blk = (VOCAB_NO_POS + VOCAB_BLK - 1) // VOCAB_BLK
    return pl.pallas_call(
        _prep_body,
        grid=(n_blk,),
        in_specs=[pl.BlockSpec((EMB, VOCAB_BLK), lambda i: (0, i))],
        out_specs=pl.BlockSpec((VOCAB_BLK, 128), lambda i: (i, 0)),
        out_shape=jax.ShapeDtypeStruct((VOCAB_NO_POS, 128), jnp.float32),
    )(table_t)


@jax.jit
def _embed(tokens_t, table_p):
    mesh = plsc.VectorSubcoreMesh(core_axis_name="c", subcore_axis_name="s")

    @functools.partial(
        pl.kernel,
        mesh=mesh,
        compiler_params=pltpu.CompilerParams(
            use_tc_tiling_on_sc=True, needs_layout_passes=False
        ),
        out_type=jax.ShapeDtypeStruct((SEQ, EMB, BATCH), jnp.float32),
        scratch_types=[
            pltpu.VMEM((SEQ, C), jnp.int32),
            pltpu.VMEM((C, 128), jnp.float32),
            pltpu.VMEM((C, 128), jnp.float32),
            pltpu.VMEM((EMB, C), jnp.float32),
            pltpu.VMEM((EMB, C), jnp.float32),
            pltpu.SemaphoreType.DMA,
            pltpu.SemaphoreType.DMA,
            pltpu.SemaphoreType.DMA,
            pltpu.SemaphoreType.DMA,
        ],
    )
    def body(tok_hbm, table_hbm, out_hbm, idx_v, ga, gb, oa, ob,
             sga, sgb, swa, swb):
        wid = lax.axis_index("s") * NC + lax.axis_index("c")
        col = wid * C
        pltpu.sync_copy(tok_hbm.at[:, pl.ds(col, C)], idx_v)

        def _clamp(i, carry):
            t = i // (C // LANES)
            j = i % (C // LANES)
            sl = pl.ds(j * LANES, LANES)
            idx_v[t, sl] = jnp.minimum(idx_v[t, sl], VOCAB_NO_POS - 1)
            return carry

        lax.fori_loop(0, SEQ * (C // LANES), _clamp, 0)

        def start_gather(t, buf, sem):
            pltpu.async_copy(table_hbm.at[idx_v.at[t]], buf, sem)

        def wait_gather(buf, sem):
            pltpu.make_async_copy(table_hbm.at[pl.ds(0, C)], buf, sem).wait()

        lane_iota = lax.iota(jnp.int32, LANES)
        s_bases = [lane_iota + (j * LANES) for j in range(C // LANES)]

        def transpose(gbuf, obuf):
            def _diag(k, carry):
                diag = (lane_iota + k) & (LANES - 1)
                for ei in range(EMB // LANES):
                    e_idx = diag + (ei * LANES)
                    for sj in range(C // LANES):
                        s_idx = s_bases[sj]
                        vals = plsc.load_gather(gbuf, [s_idx, e_idx])
                        plsc.store_scatter(obuf, [e_idx, s_idx], vals)
                return carry

            lax.fori_loop(0, LANES, _diag, 0)

        def start_write(t, obuf, sem):
            pltpu.async_copy(obuf, out_hbm.at[t, :, pl.ds(col, C)], sem)

        def wait_write(obuf, sem):
            pltpu.make_async_copy(obuf, out_hbm.at[0, :, pl.ds(0, C)], sem).wait()

        start_gather(0, ga, sga)
        n_pairs = SEQ // 2

        def pair_body(p, carry):
            t0 = p * 2

            wait_gather(ga, sga)
            start_gather(t0 + 1, gb, sgb)

            @pl.when(p > 0)
            def _():
                wait_write(oa, swa)

            transpose(ga, oa)
            start_write(t0, oa, swa)

            wait_gather(gb, sgb)

            @pl.when(p < n_pairs - 1)
            def _():
                start_gather(t0 + 2, ga, sga)

            @pl.when(p > 0)
            def _():
                wait_write(ob, swb)

            transpose(gb, ob)
            start_write(t0 + 1, ob, swb)
            return carry

        lax.fori_loop(0, n_pairs, pair_body, 0)
        wait_write(oa, swa)
        wait_write(ob, swb)

    return body(tokens_t, table_p)


def kernel(tokens, table):
    table_p = _prep_table(table.T)
    out = _embed(tokens.T.astype(jnp.int32), table_p)
    return out.transpose(2, 0, 1)


# speedup vs baseline: 1.3136x; 1.3136x over previous
import functools

import jax
import jax.numpy as jnp
from jax import lax
from jax.experimental import pallas as pl
from jax.experimental.pallas import tpu as pltpu
from jax.experimental.pallas import tpu_sc as plsc

VOCAB_NO_POS = 999001
EMB = 64
SCALE = 8.0
LANES = 16
SEQ = 200
BATCH = 4096
C = 128

_info = plsc.get_sparse_core_info()
NC = _info.num_cores
NS = _info.num_subcores
NW = NC * NS

VOCAB_BLK = 4096
FOLD = ((VOCAB_NO_POS + 1) // 2 + VOCAB_BLK - 1) // VOCAB_BLK * VOCAB_BLK


def _prep_body(a_ref, b_ref, o_ref):
    o_ref[:, 0:EMB] = a_ref[...].T * SCALE
    o_ref[:, EMB:128] = b_ref[...].T * SCALE


@jax.jit
def _prep_table(table_t):
    n_blk = FOLD // VOCAB_BLK
    return pl.pallas_call(
        _prep_body,
        grid=(n_blk,),
        in_specs=[
            pl.BlockSpec((EMB, VOCAB_BLK), lambda i: (0, i)),
            pl.BlockSpec((EMB, VOCAB_BLK), lambda i: (0, i + FOLD // VOCAB_BLK)),
        ],
        out_specs=pl.BlockSpec((VOCAB_BLK, 128), lambda i: (i, 0)),
        out_shape=jax.ShapeDtypeStruct((FOLD, 128), jnp.float32),
        compiler_params=pltpu.CompilerParams(
            dimension_semantics=("parallel",)
        ),
    )(table_t, table_t)


@jax.jit
def _embed(tokens_t, table_p):
    mesh = plsc.VectorSubcoreMesh(core_axis_name="c", subcore_axis_name="s")

    @functools.partial(
        pl.kernel,
        mesh=mesh,
        compiler_params=pltpu.CompilerParams(
            use_tc_tiling_on_sc=True, needs_layout_passes=False
        ),
        out_type=jax.ShapeDtypeStruct((SEQ, EMB, BATCH), jnp.float32),
        scratch_types=[
            pltpu.VMEM((SEQ, C), jnp.int32),
            pltpu.VMEM((SEQ, C), jnp.int32),
            pltpu.VMEM((C, 128), jnp.float32),
            pltpu.VMEM((C, 128), jnp.float32),
            pltpu.VMEM((EMB, C), jnp.float32),
            pltpu.VMEM((EMB, C), jnp.float32),
            pltpu.SemaphoreType.DMA,
            pltpu.SemaphoreType.DMA,
            pltpu.SemaphoreType.DMA,
            pltpu.SemaphoreType.DMA,
        ],
    )
    def body(tok_hbm, table_hbm, out_hbm, idx_v, off_v, ga, gb, oa, ob,
             sga, sgb, swa, swb):
        wid = lax.axis_index("s") * NC + lax.axis_index("c")
        col = wid * C
        pltpu.sync_copy(tok_hbm.at[:, pl.ds(col, C)], idx_v)

        def _clamp(i, carry):
            t = i // (C // LANES)
            j = i % (C // LANES)
            sl = pl.ds(j * LANES, LANES)
            v = jnp.minimum(idx_v[t, sl], VOCAB_NO_POS - 1)
            m = (v >= FOLD).astype(jnp.int32)
            off_v[t, sl] = m * EMB
            idx_v[t, sl] = v - m * FOLD
            return carry

        lax.fori_loop(0, SEQ * (C // LANES), _clamp, 0)

        def start_gather(t, buf, sem):
            pltpu.async_copy(table_hbm.at[idx_v.at[t]], buf, sem)

        def wait_gather(buf, sem):
            pltpu.make_async_copy(table_hbm.at[pl.ds(0, C)], buf, sem).wait()

        lane_iota = lax.iota(jnp.int32, LANES)
        s_bases = [lane_iota + (j * LANES) for j in range(C // LANES)]

        def transpose(t, gbuf, obuf):
            offs = [off_v[t, pl.ds(j * LANES, LANES)]
                    for j in range(C // LANES)]

            def _diag(k, carry):
                diag = (lane_iota + k) & (LANES - 1)
                for ei in range(EMB // LANES):
                    e_st = diag + (ei * LANES)
                    for sj in range(C // LANES):
                        s_idx = s_bases[sj]
                        vals = plsc.load_gather(gbuf, [s_idx, e_st + offs[sj]])
                        plsc.store_scatter(obuf, [e_st, s_idx], vals)
                return carry

            lax.fori_loop(0, LANES, _diag, 0)

        def start_write(t, obuf, sem):
            pltpu.async_copy(obuf, out_hbm.at[t, :, pl.ds(col, C)], sem)

        def wait_write(obuf, sem):
            pltpu.make_async_copy(obuf, out_hbm.at[0, :, pl.ds(0, C)], sem).wait()

        start_gather(0, ga, sga)
        n_pairs = SEQ // 2

        def pair_body(p, carry):
            t0 = p * 2

            wait_gather(ga, sga)
            start_gather(t0 + 1, gb, sgb)

            @pl.when(p > 0)
            def _():
                wait_write(oa, swa)

            transpose(t0, ga, oa)
            start_write(t0, oa, swa)

            wait_gather(gb, sgb)

            @pl.when(p < n_pairs - 1)
            def _():
                start_gather(t0 + 2, ga, sga)

            @pl.when(p > 0)
            def _():
                wait_write(ob, swb)

            transpose(t0 + 1, gb, ob)
            start_write(t0 + 1, ob, swb)
            return carry

        lax.fori_loop(0, n_pairs, pair_body, 0)
        wait_write(oa, swa)
        wait_write(ob, swb)

    return body(tokens_t, table_p)


def kernel(tokens, table):
    table_p = _prep_table(table.T)
    out = _embed(tokens.T.astype(jnp.int32), table_p)
    return out.transpose(2, 0, 1)


# speedup vs baseline: 1.3795x; 1.0502x over previous
import functools

import jax
import jax.numpy as jnp
from jax import lax
from jax.experimental import pallas as pl
from jax.experimental.pallas import tpu as pltpu
from jax.experimental.pallas import tpu_sc as plsc

VOCAB_NO_POS = 999001
EMB = 64
SCALE = 8.0
LANES = 16
SEQ = 200
BATCH = 4096
C = 128

_info = plsc.get_sparse_core_info()
NC = _info.num_cores
NS = _info.num_subcores
NW = NC * NS

VOCAB_BLK = 8192
FOLD = ((VOCAB_NO_POS + 1) // 2 + VOCAB_BLK - 1) // VOCAB_BLK * VOCAB_BLK


def _prep_body(a_ref, b_ref, o_ref):
    o_ref[:, 0:EMB] = a_ref[...].T * SCALE
    o_ref[:, EMB:128] = b_ref[...].T * SCALE


@jax.jit
def _prep_table(table_t):
    n_blk = FOLD // VOCAB_BLK
    return pl.pallas_call(
        _prep_body,
        grid=(n_blk,),
        in_specs=[
            pl.BlockSpec((EMB, VOCAB_BLK), lambda i: (0, i)),
            pl.BlockSpec((EMB, VOCAB_BLK), lambda i: (0, i + FOLD // VOCAB_BLK)),
        ],
        out_specs=pl.BlockSpec((VOCAB_BLK, 128), lambda i: (i, 0)),
        out_shape=jax.ShapeDtypeStruct((FOLD, 128), jnp.float32),
        compiler_params=pltpu.CompilerParams(
            dimension_semantics=("parallel",)
        ),
    )(table_t, table_t)


@jax.jit
def _embed(tokens_t, table_p):
    mesh = plsc.VectorSubcoreMesh(core_axis_name="c", subcore_axis_name="s")

    @functools.partial(
        pl.kernel,
        mesh=mesh,
        compiler_params=pltpu.CompilerParams(
            use_tc_tiling_on_sc=True, needs_layout_passes=False
        ),
        out_type=jax.ShapeDtypeStruct((SEQ, EMB, BATCH), jnp.float32),
        scratch_types=[
            pltpu.VMEM((SEQ, C), jnp.int32),
            pltpu.VMEM((SEQ, C), jnp.int32),
            pltpu.VMEM((C, 128), jnp.float32),
            pltpu.VMEM((C, 128), jnp.float32),
            pltpu.VMEM((EMB, C), jnp.float32),
            pltpu.VMEM((EMB, C), jnp.float32),
            pltpu.SemaphoreType.DMA,
            pltpu.SemaphoreType.DMA,
            pltpu.SemaphoreType.DMA,
            pltpu.SemaphoreType.DMA,
        ],
    )
    def body(tok_hbm, table_hbm, out_hbm, idx_v, off_v, ga, gb, oa, ob,
             sga, sgb, swa, swb):
        wid = lax.axis_index("s") * NC + lax.axis_index("c")
        col = wid * C
        pltpu.sync_copy(tok_hbm.at[:, pl.ds(col, C)], idx_v)

        def _clamp(i, carry):
            t = i // (C // LANES)
            j = i % (C // LANES)
            sl = pl.ds(j * LANES, LANES)
            v = jnp.minimum(idx_v[t, sl], VOCAB_NO_POS - 1)
            m = (v >= FOLD).astype(jnp.int32)
            off_v[t, sl] = m * EMB
            idx_v[t, sl] = v - m * FOLD
            return carry

        lax.fori_loop(0, SEQ * (C // LANES), _clamp, 0)

        def start_gather(t, buf, sem):
            pltpu.async_copy(table_hbm.at[idx_v.at[t]], buf, sem)

        def wait_gather(buf, sem):
            pltpu.make_async_copy(table_hbm.at[pl.ds(0, C)], buf, sem).wait()

        lane_iota = lax.iota(jnp.int32, LANES)
        s_bases = [lane_iota + (j * LANES) for j in range(C // LANES)]

        def transpose(t, gbuf, obuf):
            offs = [off_v[t, pl.ds(j * LANES, LANES)]
                    for j in range(C // LANES)]

            def _diag(k, carry):
                diag = (lane_iota + k) & (LANES - 1)
                for ei in range(EMB // LANES):
                    e_st = diag + (ei * LANES)
                    for sj in range(C // LANES):
                        s_idx = s_bases[sj]
                        vals = plsc.load_gather(gbuf, [s_idx, e_st + offs[sj]])
                        plsc.store_scatter(obuf, [e_st, s_idx], vals)
                return carry

            lax.fori_loop(0, LANES, _diag, 0)

        def start_write(t, obuf, sem):
            pltpu.async_copy(obuf, out_hbm.at[t, :, pl.ds(col, C)], sem)

        def wait_write(obuf, sem):
            pltpu.make_async_copy(obuf, out_hbm.at[0, :, pl.ds(0, C)], sem).wait()

        start_gather(0, ga, sga)
        n_pairs = SEQ // 2

        def pair_body(p, carry):
            t0 = p * 2

            wait_gather(ga, sga)
            start_gather(t0 + 1, gb, sgb)

            @pl.when(p > 0)
            def _():
                wait_write(oa, swa)

            transpose(t0, ga, oa)
            start_write(t0, oa, swa)

            wait_gather(gb, sgb)

            @pl.when(p < n_pairs - 1)
            def _():
                start_gather(t0 + 2, ga, sga)

            @pl.when(p > 0)
            def _():
                wait_write(ob, swb)

            transpose(t0 + 1, gb, ob)
            start_write(t0 + 1, ob, swb)
            return carry

        lax.fori_loop(0, n_pairs, pair_body, 0)
        wait_write(oa, swa)
        wait_write(ob, swb)

    return body(tokens_t, table_p)


def kernel(tokens, table):
    table_p = _prep_table(table.T)
    out = _embed(tokens.T.astype(jnp.int32), table_p)
    return out.transpose(2, 0, 1)
